# baseline (speedup 1.0000x reference)
import jax
import jax.numpy as jnp
from jax import lax
from jax.experimental import pallas as pl
from jax.experimental.pallas import tpu as pltpu
from jax.experimental.pallas import tpu_sc as plsc

FW = 256
NG = FW // 16
R = 32

_MESH = plsc.VectorSubcoreMesh(core_axis_name="core", subcore_axis_name="subcore")


@jax.jit
def kernel(x):
    b, s, f = x.shape
    n_chunks = s // R
    n_tasks = (f // FW) * b

    @pl.kernel(
        out_type=jax.ShapeDtypeStruct((b, s, f), x.dtype),
        mesh=_MESH,
        scratch_types=[pltpu.VMEM((FW,), x.dtype)],
    )
    def sc_cumsum(x_hbm, o_hbm, carry_ref):
        for g in range(NG):
            carry_ref[pl.ds(16 * g, 16)] = jnp.zeros((16,), x.dtype)

        def body(x_vmem, o_vmem):
            def row(r, carry):
                new = []
                for g in range(NG):
                    cg = carry[g]
                    o_vmem[0, r, pl.ds(16 * g, 16)] = cg
                    new.append(cg + x_vmem[0, r, pl.ds(16 * g, 16)])
                return tuple(new)

            c0 = tuple(carry_ref[pl.ds(16 * g, 16)] for g in range(NG))
            cn = lax.fori_loop(0, R, row, c0, unroll=4)
            for g in range(NG):
                carry_ref[pl.ds(16 * g, 16)] = cn[g]

        pltpu.emit_pipeline(
            body,
            grid=(n_tasks, n_chunks),
            in_specs=[
                pl.BlockSpec((1, R, FW), index_map=lambda t, k: (t // 8, k, t % 8)),
            ],
            out_specs=[
                pl.BlockSpec((1, R, FW), index_map=lambda t, k: (t // 8, k, t % 8)),
            ],
            core_axis_name=("core", "subcore"),
            dimension_semantics=(pltpu.PARALLEL, pltpu.ARBITRARY),
        )(x_hbm, o_hbm)

    return sc_cumsum(x)


# speedup vs baseline: 1.1948x; 1.1948x over previous
import jax
import jax.numpy as jnp
from jax import lax
from jax.experimental import pallas as pl
from jax.experimental.pallas import tpu as pltpu
from jax.experimental.pallas import tpu_sc as plsc

FW = 256
NG = FW // 16
R = 64

_MESH = plsc.VectorSubcoreMesh(core_axis_name="core", subcore_axis_name="subcore")


@jax.jit
def kernel(x):
    b, s, f = x.shape
    n_chunks = s // R
    n_tasks = (f // FW) * b

    @pl.kernel(
        out_type=jax.ShapeDtypeStruct((b, s, f), x.dtype),
        mesh=_MESH,
        scratch_types=[pltpu.VMEM((FW,), x.dtype)],
    )
    def sc_cumsum(x_hbm, o_hbm, carry_ref):
        for g in range(NG):
            carry_ref[pl.ds(16 * g, 16)] = jnp.zeros((16,), x.dtype)

        def body(x_vmem, o_vmem):
            def row(r, carry):
                new = []
                for g in range(NG):
                    cg = carry[g]
                    o_vmem[0, r, pl.ds(16 * g, 16)] = cg
                    new.append(cg + x_vmem[0, r, pl.ds(16 * g, 16)])
                return tuple(new)

            c0 = tuple(carry_ref[pl.ds(16 * g, 16)] for g in range(NG))
            cn = lax.fori_loop(0, R, row, c0, unroll=4)
            for g in range(NG):
                carry_ref[pl.ds(16 * g, 16)] = cn[g]

        pltpu.emit_pipeline(
            body,
            grid=(n_tasks, n_chunks),
            in_specs=[
                pl.BlockSpec((1, R, FW), index_map=lambda t, k: (t // 8, k, t % 8)),
            ],
            out_specs=[
                pl.BlockSpec((1, R, FW), index_map=lambda t, k: (t // 8, k, t % 8)),
            ],
            core_axis_name=("core", "subcore"),
            dimension_semantics=(pltpu.PARALLEL, pltpu.ARBITRARY),
        )(x_hbm, o_hbm)

    return sc_cumsum(x)


# speedup vs baseline: 1.1998x; 1.0042x over previous
import jax
import jax.numpy as jnp
from jax import lax
from jax.experimental import pallas as pl
from jax.experimental.pallas import tpu as pltpu
from jax.experimental.pallas import tpu_sc as plsc

FW = 256
NG = FW // 16
R = 64

_MESH = plsc.VectorSubcoreMesh(core_axis_name="core", subcore_axis_name="subcore")


@jax.jit
def kernel(x):
    b, s, f = x.shape
    n_chunks = s // R
    n_tasks = (f // FW) * b

    @pl.kernel(
        out_type=jax.ShapeDtypeStruct((b, s, f), x.dtype),
        mesh=_MESH,
        scratch_types=[pltpu.VMEM((FW,), x.dtype)],
    )
    def sc_cumsum(x_hbm, o_hbm, carry_ref):
        for g in range(NG):
            carry_ref[pl.ds(16 * g, 16)] = jnp.zeros((16,), x.dtype)

        def body(x_vmem, o_vmem):
            def row(r, carry):
                new = []
                for g in range(NG):
                    cg = carry[g]
                    o_vmem[0, r, pl.ds(16 * g, 16)] = cg
                    new.append(cg + x_vmem[0, r, pl.ds(16 * g, 16)])
                return tuple(new)

            c0 = tuple(carry_ref[pl.ds(16 * g, 16)] for g in range(NG))
            cn = lax.fori_loop(0, R, row, c0, unroll=8)
            for g in range(NG):
                carry_ref[pl.ds(16 * g, 16)] = cn[g]

        pltpu.emit_pipeline(
            body,
            grid=(n_tasks, n_chunks),
            in_specs=[
                pl.BlockSpec((1, R, FW), index_map=lambda t, k: (t // 8, k, t % 8)),
            ],
            out_specs=[
                pl.BlockSpec((1, R, FW), index_map=lambda t, k: (t // 8, k, t % 8)),
            ],
            core_axis_name=("core", "subcore"),
            dimension_semantics=(pltpu.PARALLEL, pltpu.ARBITRARY),
        )(x_hbm, o_hbm)

    return sc_cumsum(x)
